# Initial kernel scaffold; baseline (speedup 1.0000x reference)
import jax
import jax.numpy as jnp
from jax.experimental import pallas as pl


def kernel(x, edge_index, params):
    raise NotImplementedError("write your pallas kernel here")


# speedup vs baseline: 21.9763x; 21.9763x over previous
import functools

import jax
import jax.numpy as jnp
from jax import lax
from jax.experimental import pallas as pl
from jax.experimental.pallas import tpu as pltpu
from jax.experimental.pallas import tpu_sc as plsc

N = 10000
NPAD = N + 16
E_REAL = 330000
NW = 32
CHUNK = 128
NCH = 81
PW = NCH * CHUNK
E_PAD = NW * PW
RPT = 624
HID = 64

_f32 = jnp.float32
_i32 = jnp.int32


def _sds(shape, dtype=_f32):
    return jax.ShapeDtypeStruct(shape, dtype)


def _layer_mm(x, W, Ws, Wal, Wad, KC, CW):
    n, din = x.shape
    F = W.shape[1]
    BN = 2000
    nb = n // BN

    def body(x_ref, w_ref, ws_ref, wal_ref, wad_ref, *outs):
        xb = x_ref[...]
        h = jnp.dot(xb, w_ref[...], preferred_element_type=_f32)
        for c in range(KC):
            outs[c][...] = h[:, c * CW:(c + 1) * CW]
        outs[KC][...] = jnp.dot(xb, ws_ref[...], preferred_element_type=_f32)
        outs[KC + 1][...] = jnp.dot(xb, wal_ref[...], preferred_element_type=_f32)
        outs[KC + 2][...] = jnp.dot(xb, wad_ref[...], preferred_element_type=_f32)

    out_shapes = ([_sds((n, CW)) for _ in range(KC)]
                  + [_sds((n, F)), _sds((n, 8)), _sds((n, 8))])
    out_specs = ([pl.BlockSpec((BN, CW), lambda i: (i, 0)) for _ in range(KC)]
                 + [pl.BlockSpec((BN, F), lambda i: (i, 0)),
                    pl.BlockSpec((BN, 8), lambda i: (i, 0)),
                    pl.BlockSpec((BN, 8), lambda i: (i, 0))])
    in_specs = [pl.BlockSpec((BN, din), lambda i: (i, 0)),
                pl.BlockSpec((din, F), lambda i: (0, 0)),
                pl.BlockSpec((din, F), lambda i: (0, 0)),
                pl.BlockSpec((din, 8), lambda i: (0, 0)),
                pl.BlockSpec((din, 8), lambda i: (0, 0))]
    return pl.pallas_call(
        body, grid=(nb,), in_specs=in_specs, out_specs=out_specs,
        out_shape=out_shapes)(x, W, Ws, Wal, Wad)


def _post_gat(conv_p, den_p, skipm, Mexp, bvec, bsvec, gvec, bevec, KC, CW):
    n = skipm.shape[0]

    def body(cp_ref, den_ref, sk_ref, m_ref, b_ref, bs_ref, g_ref, be_ref, o_ref):
        d8 = den_ref[0] + den_ref[1]
        dinv = 1.0 / (d8 + 1e-16)
        dcols = jnp.dot(dinv, m_ref[...], preferred_element_type=_f32)
        cp = cp_ref[0, 0] + cp_ref[1, 0]
        y = cp * dcols + sk_ref[...] + b_ref[...] + bs_ref[...]
        mu = jnp.mean(y, axis=0, keepdims=True)
        var = jnp.mean((y - mu) ** 2, axis=0, keepdims=True)
        z = (y - mu) / jnp.sqrt(var + 1e-5) * g_ref[...] + be_ref[...]
        o_ref[...] = jnp.where(z > 0, z, jnp.exp(jnp.minimum(z, 0.0)) - 1.0)

    return pl.pallas_call(
        body, grid=(KC,),
        in_specs=[pl.BlockSpec((2, 1, n, CW), lambda j: (0, j, 0, 0)),
                  pl.BlockSpec((2, n, 8), lambda j: (0, 0, 0)),
                  pl.BlockSpec((n, CW), lambda j: (0, j)),
                  pl.BlockSpec((8, CW), lambda j: (0, j)),
                  pl.BlockSpec((1, CW), lambda j: (0, j)),
                  pl.BlockSpec((1, CW), lambda j: (0, j)),
                  pl.BlockSpec((1, CW), lambda j: (0, j)),
                  pl.BlockSpec((1, CW), lambda j: (0, j))],
        out_specs=pl.BlockSpec((n, CW), lambda j: (0, j)),
        out_shape=_sds((n, KC * CW)),
    )(conv_p, den_p, skipm, Mexp, bvec, bsvec, gvec, bevec)


def _gcn_premm(x, Wg, deg_p):
    n, din = x.shape
    BN = 2000
    nb = n // BN

    def body(x_ref, w_ref, deg_ref, o_ref):
        deg = deg_ref[0, :, 0:1] + deg_ref[1, :, 0:1]
        dinv = jnp.where(deg > 0, lax.rsqrt(deg), 0.0)
        o_ref[...] = jnp.dot(x_ref[...], w_ref[...],
                             preferred_element_type=_f32) * dinv

    return pl.pallas_call(
        body, grid=(nb,),
        in_specs=[pl.BlockSpec((BN, din), lambda i: (i, 0)),
                  pl.BlockSpec((din, 32), lambda i: (0, 0)),
                  pl.BlockSpec((2, BN, 8), lambda i: (0, i, 0))],
        out_specs=pl.BlockSpec((BN, 32), lambda i: (i, 0)),
        out_shape=_sds((n, 32)))(x, Wg, deg_p)


def _final_head(outg_p, deg_p, bg, Wm1, bm1, gm, bem, Wm2, bm2):
    n = N

    def body(og_ref, deg_ref, bg_ref, w1_ref, b1_ref, g_ref, be_ref,
             w2_ref, b2_ref, o_ref):
        deg = deg_ref[0, :, 0:1] + deg_ref[1, :, 0:1]
        dinv = jnp.where(deg > 0, lax.rsqrt(deg), 0.0)
        x1 = (og_ref[0, 0] + og_ref[1, 0]) * dinv + bg_ref[...]
        z = jnp.dot(x1, w1_ref[...], preferred_element_type=_f32) + b1_ref[...]
        mu = jnp.mean(z, axis=0, keepdims=True)
        var = jnp.mean((z - mu) ** 2, axis=0, keepdims=True)
        z = (z - mu) / jnp.sqrt(var + 1e-5) * g_ref[...] + be_ref[...]
        z = jnp.maximum(z, 0.0)
        z2 = jnp.dot(z, w2_ref[...], preferred_element_type=_f32) + b2_ref[...]
        m = jnp.max(z2, axis=1, keepdims=True)
        lse = jnp.log(jnp.sum(jnp.exp(z2 - m), axis=1, keepdims=True)) + m
        o_ref[...] = z2 - lse

    return pl.pallas_call(
        body, out_shape=_sds((n, 32)),
    )(outg_p, deg_p, bg, Wm1, bm1, gm, bem, Wm2, bm2)


def _sc_mesh():
    return plsc.VectorSubcoreMesh(core_axis_name="c", subcore_axis_name="s",
                                  num_cores=2, num_subcores=16)


def _fill16(buf, nrows, ncols, val):
    v = jnp.full((16,), val, _f32)

    def row(r, _):
        for t in range(ncols // 16):
            buf[r, pl.ds(t * 16, 16)] = v
        return 0

    lax.fori_loop(0, nrows, row, 0)


def _fill8(buf, nrows, val, rowoff, coloff):
    v = jnp.full((16,), val, _f32)

    def row(r, _):
        plsc.store_scatter(buf, [2 * r + rowoff, coloff], v)
        return 0

    lax.fori_loop(0, nrows // 2, row, 0)


def _zero_rows(zsrc, sp_ref, s):
    for off, sz in ((0, 128), (128, 128), (256, 128), (384, 128), (512, 112)):
        pltpu.sync_copy(zsrc.at[pl.ds(0, sz)],
                        sp_ref.at[pl.ds(s * RPT + off, sz)])

    @pl.when(s == 15)
    def _():
        pltpu.sync_copy(zsrc.at[pl.ds(0, 16)], sp_ref.at[pl.ds(16 * RPT, 16)])


def _dump_rows(sp_ref, make_dst, s):
    pltpu.sync_copy(sp_ref.at[pl.ds(s * RPT, RPT)], make_dst(s * RPT, RPT))

    @pl.when(s == 15)
    def _():
        pltpu.sync_copy(sp_ref.at[pl.ds(16 * RPT, 16)], make_dst(16 * RPT, 16))


def _build_idx(dstv, idxw, k, lane, wbase):
    for t in range(8):
        dv = dstv[pl.ds(k * CHUNK + t * 16, 16)]
        eid = wbase + k * CHUNK + t * 16 + lane
        idxw[pl.ds(t * 16, 16)] = jnp.where(eid < E_REAL, dv, N)


def _gat_sc(KC, CW, with_deg):
    HPC = CW // HID

    out_type = [_sds((2, N, 8)), _sds((2, KC, N, CW)), _sds((E_PAD, 8))]
    if with_deg:
        out_type.append(_sds((2, N, 8)))

    scratch = [
        pltpu.VMEM((PW,), _i32),
        pltpu.VMEM((PW,), _i32),
        pltpu.VMEM((CHUNK,), _i32),
        pltpu.VMEM((CHUNK, 8), _f32),
        pltpu.VMEM((CHUNK, 8), _f32),
        pltpu.VMEM((CHUNK, 8), _f32),
        pltpu.VMEM((CHUNK, CW), _f32),
        pltpu.VMEM((CHUNK, 8), _f32),
        pltpu.VMEM_SHARED((NPAD, 8), _f32),
        pltpu.VMEM_SHARED((NPAD, CW), _f32),
        pltpu.SemaphoreType.DMA,
        pltpu.SemaphoreType.DMA,
    ]
    if with_deg:
        scratch.append(pltpu.VMEM_SHARED((NPAD, 8), _f32))

    def body(als_t, ald_t, *rest):
        h_tabs = rest[:KC]
        srch, dsth = rest[KC:KC + 2]
        pos = KC + 2
        den_hbm, outp_hbm, ex_hbm = rest[pos:pos + 3]
        pos += 3
        if with_deg:
            deg_hbm = rest[pos]; pos += 1
        (srcv, dstv, idxw, bufS, bufD, exb, hbuf, onesb,
         den_sp, out_sp, sem0, sem1) = rest[pos:pos + 12]
        if with_deg:
            deg_sp = rest[pos + 12]

        c = lax.axis_index("c")
        s = lax.axis_index("s")
        wid = s * 2 + c
        wbase = wid * PW
        lane = lax.iota(_i32, 16)
        rowoff = lane >> 3
        coloff = lane & 7

        pltpu.sync_copy(srch.at[wid], srcv)
        pltpu.sync_copy(dsth.at[wid], dstv)
        _fill8(onesb, CHUNK, 1.0, rowoff, coloff)
        _fill8(exb, CHUNK, 0.0, rowoff, coloff)
        _zero_rows(exb, den_sp, s)
        if with_deg:
            _zero_rows(exb, deg_sp, s)
        plsc.subcore_barrier()

        def den_chunk(k, _):
            sidx = srcv.at[pl.ds(k * CHUNK, CHUNK)]
            didx = dstv.at[pl.ds(k * CHUNK, CHUNK)]
            cp0 = pltpu.async_copy(als_t.at[sidx], bufS, sem0)
            cp1 = pltpu.async_copy(ald_t.at[didx], bufD, sem1)
            cp0.wait()
            cp1.wait()

            def row(r, _):
                ri = 2 * r + rowoff
                sv = plsc.load_gather(bufS, [ri, coloff])
                dv = plsc.load_gather(bufD, [ri, coloff])
                e = sv + dv
                e = jnp.where(e >= 0, e, 0.2 * e)
                ex = jnp.exp(e)
                plsc.store_scatter(exb, [ri, coloff], ex)
                return 0

            lax.fori_loop(0, CHUNK // 2, row, 0)
            _build_idx(dstv, idxw, k, lane, wbase)
            pltpu.sync_copy(exb, ex_hbm.at[pl.ds(wbase + k * CHUNK, CHUNK)])
            pltpu.sync_copy(exb, den_sp.at[idxw], add=True)
            if with_deg:
                pltpu.sync_copy(onesb, deg_sp.at[idxw], add=True)
            return 0

        lax.fori_loop(0, NCH, den_chunk, 0)
        plsc.subcore_barrier()
        _dump_rows(den_sp, lambda r, m: den_hbm.at[c, pl.ds(r, m)], s)
        if with_deg:
            _dump_rows(deg_sp, lambda r, m: deg_hbm.at[c, pl.ds(r, m)], s)

        for cc in range(KC):
            _fill16(hbuf, CHUNK, CW, 0.0)
            _zero_rows(hbuf, out_sp, s)
            plsc.subcore_barrier()

            def agg_chunk(k, _):
                sidx = srcv.at[pl.ds(k * CHUNK, CHUNK)]
                pltpu.async_copy(h_tabs[cc].at[sidx], hbuf, sem0).wait()
                pltpu.sync_copy(ex_hbm.at[pl.ds(wbase + k * CHUNK, CHUNK)], exb)

                def edge(e, _):
                    ei = jnp.broadcast_to(e, (16,)).astype(_i32)
                    for hh in range(HPC):
                        hcol = jnp.broadcast_to(HPC * cc + hh, (16,)).astype(_i32)
                        w = plsc.load_gather(exb, [ei, hcol])
                        for t in range(HID // 16):
                            col = hh * HID + t * 16
                            hbuf[e, pl.ds(col, 16)] = hbuf[e, pl.ds(col, 16)] * w
                    return 0

                lax.fori_loop(0, CHUNK, edge, 0)
                _build_idx(dstv, idxw, k, lane, wbase)
                pltpu.sync_copy(hbuf, out_sp.at[idxw], add=True)
                return 0

            lax.fori_loop(0, NCH, agg_chunk, 0)
            plsc.subcore_barrier()
            _dump_rows(out_sp, lambda r, m: outp_hbm.at[c, cc, pl.ds(r, m)], s)
            plsc.subcore_barrier()
        return None

    return functools.partial(pl.kernel, body, out_type=tuple(out_type),
                             mesh=_sc_mesh(), scratch_types=tuple(scratch),
                             compiler_params=pltpu.CompilerParams(
                                 needs_layout_passes=False,
                                 use_tc_tiling_on_sc=False))


def _gcn_sc():
    CW = 32
    out_type = (_sds((2, 1, N, CW)),)
    scratch = [
        pltpu.VMEM((PW,), _i32),
        pltpu.VMEM((PW,), _i32),
        pltpu.VMEM((CHUNK,), _i32),
        pltpu.VMEM((CHUNK, CW), _f32),
        pltpu.VMEM_SHARED((NPAD, CW), _f32),
        pltpu.SemaphoreType.DMA,
    ]

    def body(h_tab, srch, dsth, outp_hbm,
             srcv, dstv, idxw, hbuf, out_sp, sem0):
        c = lax.axis_index("c")
        s = lax.axis_index("s")
        wid = s * 2 + c
        wbase = wid * PW
        lane = lax.iota(_i32, 16)

        pltpu.sync_copy(srch.at[wid], srcv)
        pltpu.sync_copy(dsth.at[wid], dstv)
        _fill16(hbuf, CHUNK, CW, 0.0)
        _zero_rows(hbuf, out_sp, s)
        plsc.subcore_barrier()

        def chunk(k, _):
            sidx = srcv.at[pl.ds(k * CHUNK, CHUNK)]
            pltpu.async_copy(h_tab.at[sidx], hbuf, sem0).wait()
            _build_idx(dstv, idxw, k, lane, wbase)
            pltpu.sync_copy(hbuf, out_sp.at[idxw], add=True)
            return 0

        lax.fori_loop(0, NCH, chunk, 0)
        plsc.subcore_barrier()
        _dump_rows(out_sp, lambda r, m: outp_hbm.at[c, 0, pl.ds(r, m)], s)
        return None

    return functools.partial(pl.kernel, body, out_type=out_type,
                             mesh=_sc_mesh(), scratch_types=tuple(scratch),
                             compiler_params=pltpu.CompilerParams(
                                 needs_layout_passes=False,
                                 use_tc_tiling_on_sc=False))


_LAYER_KC_CW = [(4, 128), (4, 128), (4, 128), (1, 64)]
_HEADS = [8, 8, 8, 1]


def kernel(x, edge_index, params):
    n = N
    loop = jnp.arange(n, dtype=edge_index.dtype)
    src = jnp.concatenate([edge_index[0], loop,
                           jnp.zeros((E_PAD - E_REAL,), edge_index.dtype)])
    dst = jnp.concatenate([edge_index[1], loop,
                           jnp.zeros((E_PAD - E_REAL,), edge_index.dtype)])
    src = src.astype(_i32).reshape(NW, PW)
    dst = dst.astype(_i32).reshape(NW, PW)

    deg_p = None
    for i in range(4):
        KC, CW = _LAYER_KC_CW[i]
        H = _HEADS[i]
        W = params[f"W{i}"]
        a_s, a_d = params[f"as{i}"], params[f"ad{i}"]
        din = W.shape[0]
        Wal = (W.reshape(din, H, HID) * a_s[None]).sum(-1)
        Wad = (W.reshape(din, H, HID) * a_d[None]).sum(-1)
        if H < 8:
            Wal = jnp.pad(Wal, ((0, 0), (0, 8 - H)))
            Wad = jnp.pad(Wad, ((0, 0), (0, 8 - H)))

        outs = _layer_mm(x, W, params[f"Ws{i}"], Wal, Wad, KC, CW)
        h_tabs, skipm, als_t, ald_t = outs[:KC], outs[KC], outs[KC + 1], outs[KC + 2]

        with_deg = i == 3
        sc = _gat_sc(KC, CW, with_deg)()
        sc_out = sc(als_t, ald_t, *h_tabs, src, dst)
        den_p, conv_p = sc_out[0], sc_out[1]
        if with_deg:
            deg_p = sc_out[3]
        del sc_out

        F = KC * CW
        Mexp = (jnp.arange(8)[:, None]
                == (jnp.arange(F) // HID)[None, :]).astype(_f32)
        x = _post_gat(conv_p, den_p, skipm,
                      Mexp, params[f"b{i}"].reshape(1, F),
                      params[f"bs{i}"].reshape(1, F),
                      params[f"g{i}"].reshape(1, F),
                      params[f"be{i}"].reshape(1, F), KC, CW)

    hgs = _gcn_premm(x, params["Wg"], deg_p)
    outg_p = _gcn_sc()()(hgs, src, dst)[0]
    return _final_head(outg_p, deg_p, params["bg"].reshape(1, 32),
                       params["Wm1"], params["bm1"].reshape(1, HID),
                       params["gm"].reshape(1, HID),
                       params["bem"].reshape(1, HID),
                       params["Wm2"], params["bm2"].reshape(1, 32))
